# baseline (speedup 1.0000x reference)
import functools

import jax
import jax.numpy as jnp
from jax import lax
from jax.experimental import pallas as pl
from jax.experimental.pallas import tpu as pltpu
from jax.experimental.pallas import tpu_sc as plsc

_VOCAB = 50257
_D = 1024
_SEQ = 4096
_NLAYERS = 24
_NSKIP = 4

_NC = 2
_NS = 16
_NW = _NC * _NS
_ROWS_PER_W = _SEQ // _NW
_CH = 32
_NCH = _ROWS_PER_W // _CH
_NBUF = 3
_NG = _D // 16


def _accum(buf, acc_v):
    zero = jnp.zeros((16,), jnp.float32)

    def body(j, carry):
        off = 16 * j
        a = [zero, zero, zero, zero]
        for r in range(_CH):
            a[r % 4] = a[r % 4] + buf[r, pl.ds(off, 16)]
        plsc.addupdate(acc_v.at[0, pl.ds(off, 16)], (a[0] + a[1]) + (a[2] + a[3]))
        return carry

    lax.fori_loop(0, _NG, body, 0, unroll=1)


def _sc_gather_body(ids_hbm, table_hbm, hid_hbm, part_hbm,
                    idx_v, rows_a, rows_b, rows_c, acc_v,
                    sem_ga, sem_gb, sem_gc, sem_sa, sem_sb, sem_sc):
    c = lax.axis_index("c")
    s = lax.axis_index("s")
    wid = s * _NC + c
    base = wid * _ROWS_PER_W
    pltpu.sync_copy(ids_hbm.at[0, pl.ds(base, _ROWS_PER_W)], idx_v)

    bufs = (rows_a, rows_b, rows_c)
    gsems = (sem_ga, sem_gb, sem_gc)
    ssems = (sem_sa, sem_sb, sem_sc)
    gath = [
        pltpu.async_copy(table_hbm.at[idx_v.at[pl.ds(ci * _CH, _CH)]],
                         bufs[ci], gsems[ci])
        for ci in range(_NBUF)
    ]
    zf = jnp.zeros((16,), jnp.float32)
    for j in range(_NG):
        acc_v[0, pl.ds(16 * j, 16)] = zf

    scat = [None, None, None]
    for ci in range(_NCH):
        b = ci % _NBUF
        gath[b].wait()
        scat[b] = pltpu.async_copy(
            bufs[b], hid_hbm.at[pl.ds(base + ci * _CH, _CH)], ssems[b])
        if ci > 0:
            pb = (ci - 1) % _NBUF
            _accum(bufs[pb], acc_v)
            nxt = ci - 1 + _NBUF
            if nxt < _NCH:
                scat[pb].wait()
                gath[pb] = pltpu.async_copy(
                    table_hbm.at[idx_v.at[pl.ds(nxt * _CH, _CH)]],
                    bufs[pb], gsems[pb])
    _accum(bufs[(_NCH - 1) % _NBUF], acc_v)
    scat[1].wait()
    scat[2].wait()
    scat[0].wait()
    pltpu.sync_copy(acc_v, part_hbm.at[pl.ds(wid, 1)])


@functools.cache
def _sc_gather():
    return pl.kernel(
        _sc_gather_body,
        mesh=plsc.VectorSubcoreMesh(core_axis_name="c", subcore_axis_name="s"),
        out_type=[
            jax.ShapeDtypeStruct((_SEQ, _D), jnp.float32),
            jax.ShapeDtypeStruct((_NW, _D), jnp.float32),
        ],
        scratch_types=[
            pltpu.VMEM((_ROWS_PER_W,), jnp.int32),
            pltpu.VMEM((_CH, _D), jnp.float32),
            pltpu.VMEM((_CH, _D), jnp.float32),
            pltpu.VMEM((_CH, _D), jnp.float32),
            pltpu.VMEM((1, _D), jnp.float32),
            pltpu.SemaphoreType.DMA,
            pltpu.SemaphoreType.DMA,
            pltpu.SemaphoreType.DMA,
            pltpu.SemaphoreType.DMA,
            pltpu.SemaphoreType.DMA,
            pltpu.SemaphoreType.DMA,
        ],
    )


def _score_body(part_ref, w_ref, out_ref):
    pooled = jnp.sum(part_ref[...], axis=0, keepdims=True) * (1.0 / _SEQ)
    scores = jnp.sum(pooled.reshape(_D, 1) * w_ref[...], axis=0,
                     keepdims=True)
    a = jnp.abs(scores)
    idxs = lax.broadcasted_iota(jnp.int32, (1, _NLAYERS), 1)
    for k in range(_NSKIP):
        m = jnp.min(a)
        i = jnp.min(jnp.where(a <= m, idxs, jnp.int32(2**30)))
        out_ref[k] = i
        a = jnp.where(idxs == i, jnp.float32(jnp.inf), a)


def _score_topk(parts, w):
    return pl.pallas_call(
        _score_body,
        out_shape=jax.ShapeDtypeStruct((_NSKIP,), jnp.int32),
        out_specs=pl.BlockSpec(memory_space=pltpu.SMEM),
    )(parts, w)


def kernel(input_ids, table, W_score):
    ids = input_ids
    if ids.dtype != jnp.int32:
        ids = ids.astype(jnp.int32)
    hid, parts = _sc_gather()(ids, table)
    skip = _score_topk(parts, W_score)
    return hid.reshape(1, _SEQ, _D), skip


# speedup vs baseline: 1.0375x; 1.0375x over previous
import functools

import jax
import jax.numpy as jnp
from jax import lax
from jax.experimental import pallas as pl
from jax.experimental.pallas import tpu as pltpu
from jax.experimental.pallas import tpu_sc as plsc

_VOCAB = 50257
_D = 1024
_SEQ = 4096
_NLAYERS = 24
_NSKIP = 4

_NC = 2
_NS = 16
_NW = _NC * _NS
_ROWS_PER_W = _SEQ // _NW
_CH = 32
_NCH = _ROWS_PER_W // _CH
_NBUF = 3
_NG = _D // 16


def _accum(buf, acc_v):
    zero = jnp.zeros((16,), jnp.float32)

    def body(j, carry):
        off = 16 * j
        a = [zero, zero, zero, zero]
        for r in range(_CH):
            a[r % 4] = a[r % 4] + buf[r, pl.ds(off, 16)]
        plsc.addupdate(acc_v.at[0, pl.ds(off, 16)], (a[0] + a[1]) + (a[2] + a[3]))
        return carry

    lax.fori_loop(0, _NG, body, 0, unroll=1)


def _sc_gather_body(ids_hbm, table_hbm, hid_hbm, part_hbm,
                    idx_v, rows_a, rows_b, rows_c, acc_v,
                    sem_ga, sem_gb, sem_gc, sem_sa, sem_sb, sem_sc):
    c = lax.axis_index("c")
    s = lax.axis_index("s")
    wid = s * _NC + c
    base = wid * _ROWS_PER_W
    pltpu.sync_copy(ids_hbm.at[0, pl.ds(base, _ROWS_PER_W)], idx_v)

    bufs = (rows_a, rows_b, rows_c)
    gsems = (sem_ga, sem_gb, sem_gc)
    ssems = (sem_sa, sem_sb, sem_sc)
    gath = [
        pltpu.async_copy(table_hbm.at[idx_v.at[pl.ds(ci * _CH, _CH)]],
                         bufs[ci], gsems[ci])
        for ci in range(_NBUF)
    ]
    zf = jnp.zeros((16,), jnp.float32)
    for j in range(_NG):
        acc_v[0, pl.ds(16 * j, 16)] = zf

    scat = [None, None, None]
    for ci in range(_NCH):
        b = ci % _NBUF
        gath[b].wait()
        scat[b] = pltpu.async_copy(
            bufs[b], hid_hbm.at[pl.ds(base + ci * _CH, _CH)], ssems[b])
        _accum(bufs[b], acc_v)
        nxt = ci + _NBUF
        if nxt < _NCH:
            scat[b].wait()
            gath[b] = pltpu.async_copy(
                table_hbm.at[idx_v.at[pl.ds(nxt * _CH, _CH)]], bufs[b], gsems[b])
    scat[1].wait()
    scat[2].wait()
    scat[0].wait()
    pltpu.sync_copy(acc_v, part_hbm.at[pl.ds(wid, 1)])


@functools.cache
def _sc_gather():
    return pl.kernel(
        _sc_gather_body,
        mesh=plsc.VectorSubcoreMesh(core_axis_name="c", subcore_axis_name="s"),
        out_type=[
            jax.ShapeDtypeStruct((_SEQ, _D), jnp.float32),
            jax.ShapeDtypeStruct((_NW, _D), jnp.float32),
        ],
        scratch_types=[
            pltpu.VMEM((_ROWS_PER_W,), jnp.int32),
            pltpu.VMEM((_CH, _D), jnp.float32),
            pltpu.VMEM((_CH, _D), jnp.float32),
            pltpu.VMEM((_CH, _D), jnp.float32),
            pltpu.VMEM((1, _D), jnp.float32),
            pltpu.SemaphoreType.DMA,
            pltpu.SemaphoreType.DMA,
            pltpu.SemaphoreType.DMA,
            pltpu.SemaphoreType.DMA,
            pltpu.SemaphoreType.DMA,
            pltpu.SemaphoreType.DMA,
        ],
    )


def _score_body(part_ref, w_ref, out_ref):
    pooled = jnp.sum(part_ref[...], axis=0, keepdims=True) * (1.0 / _SEQ)
    scores = jnp.sum(pooled.reshape(_D, 1) * w_ref[...], axis=0,
                     keepdims=True)
    a = jnp.abs(scores)
    idxs = lax.broadcasted_iota(jnp.int32, (1, _NLAYERS), 1)
    for k in range(_NSKIP):
        m = jnp.min(a)
        i = jnp.min(jnp.where(a <= m, idxs, jnp.int32(2**30)))
        out_ref[k] = i
        a = jnp.where(idxs == i, jnp.float32(jnp.inf), a)


def _score_topk(parts, w):
    return pl.pallas_call(
        _score_body,
        out_shape=jax.ShapeDtypeStruct((_NSKIP,), jnp.int32),
        out_specs=pl.BlockSpec(memory_space=pltpu.SMEM),
    )(parts, w)


def kernel(input_ids, table, W_score):
    ids = input_ids
    if ids.dtype != jnp.int32:
        ids = ids.astype(jnp.int32)
    hid, parts = _sc_gather()(ids, table)
    skip = _score_topk(parts, W_score)
    return hid.reshape(1, _SEQ, _D), skip
